# Initial kernel scaffold; baseline (speedup 1.0000x reference)
import jax
import jax.numpy as jnp
from jax.experimental import pallas as pl


def kernel(x, senders, receivers, W1, b1, W2, b2, Wmu, bmu, Wls, bls):
    raise NotImplementedError("write your pallas kernel here")


# speedup vs baseline: 4.6554x; 4.6554x over previous
import functools

import jax
import jax.numpy as jnp
from jax import lax
from jax.experimental import pallas as pl
from jax.experimental.pallas import tpu as pltpu
from jax.experimental.pallas import tpu_sc as plsc

N_NODES = 100000
HID = 20
NSUB = 16
EB = 128
HC = 16
NP = N_NODES + HC
RPT = NP // NSUB
BN = 2000

_mesh = plsc.VectorSubcoreMesh(core_axis_name="c", subcore_axis_name="s")
_CP = pltpu.CompilerParams(use_tc_tiling_on_sc=False)


def _sc_degree(EP):
    ept = EP // NSUB
    chunks = ept // EB

    @functools.partial(
        pl.kernel,
        out_type=jax.ShapeDtypeStruct((2, NP, HC), jnp.float32),
        mesh=_mesh,
        compiler_params=_CP,
        scratch_types=[
            pltpu.VMEM((EB,), jnp.int32),
            pltpu.VMEM((EB, HC), jnp.float32),
            pltpu.VMEM_SHARED((NP, HC), jnp.float32),
        ],
    )
    def deg_kernel(s_hbm, r_hbm, ones_hbm, zeros_hbm, deg_out,
                   idx, ones_v, acc):
        c = lax.axis_index("c")
        s = lax.axis_index("s")
        r0 = s * RPT
        pltpu.sync_copy(zeros_hbm, acc.at[pl.ds(r0, RPT)])
        pltpu.sync_copy(ones_hbm, ones_v)
        plsc.subcore_barrier()

        def scan_edges(e_hbm):
            def body(g, carry):
                pltpu.sync_copy(e_hbm.at[pl.ds(s * ept + g * EB, EB)], idx)
                pltpu.sync_copy(ones_v, acc.at[idx], add=True)
                return carry
            lax.fori_loop(0, chunks, body, 0)

        @pl.when(c == 0)
        def _():
            scan_edges(s_hbm)

        @pl.when(c == 1)
        def _():
            scan_edges(r_hbm)

        plsc.subcore_barrier()
        pltpu.sync_copy(acc.at[pl.ds(r0, RPT)], deg_out.at[c, pl.ds(r0, RPT)])

    return deg_kernel


def _sc_message(EP):
    ept = EP // NSUB
    chunks = ept // EB

    @functools.partial(
        pl.kernel,
        out_type=jax.ShapeDtypeStruct((2, NP, HC), jnp.float32),
        mesh=_mesh,
        compiler_params=_CP,
        scratch_types=[
            pltpu.VMEM((EB,), jnp.int32),
            pltpu.VMEM((EB,), jnp.int32),
            pltpu.VMEM((EB, HC), jnp.float32),
            pltpu.VMEM_SHARED((NP, HC), jnp.float32),
            pltpu.SemaphoreType.DMA,
        ],
    )
    def msg_kernel(z0_hbm, z1_hbm, s_hbm, r_hbm, zeros_hbm, y_out,
                   s_idx, r_idx, rows, acc, sem):
        c = lax.axis_index("c")
        s = lax.axis_index("s")
        r0 = s * RPT
        pltpu.sync_copy(zeros_hbm, acc.at[pl.ds(r0, RPT)])
        plsc.subcore_barrier()

        def scan_edges(z_hbm):
            def body(g, carry):
                base = s * ept + g * EB
                pltpu.sync_copy(s_hbm.at[pl.ds(base, EB)], s_idx)
                pltpu.sync_copy(r_hbm.at[pl.ds(base, EB)], r_idx)
                pltpu.async_copy(z_hbm.at[s_idx], rows, sem).wait()
                pltpu.sync_copy(rows, acc.at[r_idx], add=True)
                return carry
            lax.fori_loop(0, chunks, body, 0)

        @pl.when(c == 0)
        def _():
            scan_edges(z0_hbm)

        @pl.when(c == 1)
        def _():
            scan_edges(z1_hbm)

        plsc.subcore_barrier()
        pltpu.sync_copy(acc.at[pl.ds(r0, RPT)], y_out.at[c, pl.ds(r0, RPT)])

    return msg_kernel


def _norm_softmax(h, deg_col):
    h = jnp.maximum(h, 0.0)
    e = jnp.exp(h - jnp.max(h, axis=-1, keepdims=True))
    sm = e / jnp.sum(e, axis=-1, keepdims=True)
    return sm * lax.rsqrt(jnp.maximum(deg_col, 1.0))[:, None]


def _split_cols(z):
    pad = jnp.zeros((z.shape[0], 2 * HC - HID), z.dtype)
    return z[:, :HC], jnp.concatenate([z[:, HC:], pad], axis=-1)


def _tc_layer1(x, W1, b1, deg):
    grid = (N_NODES // BN,)

    def body(x_ref, w_ref, b_ref, d_ref, o0_ref, o1_ref):
        h = jnp.dot(x_ref[...], w_ref[...], preferred_element_type=jnp.float32)
        z = _norm_softmax(h + b_ref[...], d_ref[0][:, 0])
        z0, z1 = _split_cols(z)
        o0_ref[...] = z0
        o1_ref[...] = z1

    return pl.pallas_call(
        body,
        grid=grid,
        in_specs=[
            pl.BlockSpec((BN, x.shape[1]), lambda i: (i, 0)),
            pl.BlockSpec(W1.shape, lambda i: (0, 0)),
            pl.BlockSpec((1, HID), lambda i: (0, 0)),
            pl.BlockSpec((2, BN, HC), lambda i: (0, i, 0)),
        ],
        out_specs=[
            pl.BlockSpec((BN, HC), lambda i: (i, 0)),
            pl.BlockSpec((BN, HC), lambda i: (i, 0)),
        ],
        out_shape=[
            jax.ShapeDtypeStruct((N_NODES, HC), jnp.float32),
            jax.ShapeDtypeStruct((N_NODES, HC), jnp.float32),
        ],
    )(x, W1, b1[None], deg)


def _recombine(y_ref, d_ref):
    rd = d_ref[1][:, 0]
    h = jnp.concatenate([y_ref[0], y_ref[1][:, : HID - HC]], axis=-1)
    return h * lax.rsqrt(jnp.maximum(rd, 1.0))[:, None]


def _tc_layer2(y1, W2, b2, deg):
    grid = (N_NODES // BN,)

    def body(y_ref, w_ref, b_ref, d_ref, o0_ref, o1_ref):
        h1 = _recombine(y_ref, d_ref)
        h = jnp.dot(h1, w_ref[...], preferred_element_type=jnp.float32)
        z = _norm_softmax(h + b_ref[...], d_ref[0][:, 0])
        z0, z1 = _split_cols(z)
        o0_ref[...] = z0
        o1_ref[...] = z1

    return pl.pallas_call(
        body,
        grid=grid,
        in_specs=[
            pl.BlockSpec((2, BN, HC), lambda i: (0, i, 0)),
            pl.BlockSpec(W2.shape, lambda i: (0, 0)),
            pl.BlockSpec((1, HID), lambda i: (0, 0)),
            pl.BlockSpec((2, BN, HC), lambda i: (0, i, 0)),
        ],
        out_specs=[
            pl.BlockSpec((BN, HC), lambda i: (i, 0)),
            pl.BlockSpec((BN, HC), lambda i: (i, 0)),
        ],
        out_shape=[
            jax.ShapeDtypeStruct((N_NODES, HC), jnp.float32),
            jax.ShapeDtypeStruct((N_NODES, HC), jnp.float32),
        ],
    )(y1, W2, b2[None], deg)


def _tc_heads(y2, deg, x, Wmu_h, Wmu_x, bmu, Wls_h, Wls_x, bls):
    grid = (N_NODES // BN,)
    Z = Wmu_h.shape[1]

    def body(y_ref, d_ref, x_ref, wmh_ref, wmx_ref, bm_ref,
             wlh_ref, wlx_ref, bl_ref, mu_ref, ls_ref):
        h2 = _recombine(y_ref, d_ref)
        xb = x_ref[...]
        mu_ref[...] = (jnp.dot(h2, wmh_ref[...], preferred_element_type=jnp.float32)
                       + jnp.dot(xb, wmx_ref[...], preferred_element_type=jnp.float32)
                       + bm_ref[...])
        ls_ref[...] = (jnp.dot(h2, wlh_ref[...], preferred_element_type=jnp.float32)
                       + jnp.dot(xb, wlx_ref[...], preferred_element_type=jnp.float32)
                       + bl_ref[...])

    return pl.pallas_call(
        body,
        grid=grid,
        in_specs=[
            pl.BlockSpec((2, BN, HC), lambda i: (0, i, 0)),
            pl.BlockSpec((2, BN, HC), lambda i: (0, i, 0)),
            pl.BlockSpec((BN, x.shape[1]), lambda i: (i, 0)),
            pl.BlockSpec(Wmu_h.shape, lambda i: (0, 0)),
            pl.BlockSpec(Wmu_x.shape, lambda i: (0, 0)),
            pl.BlockSpec((1, Z), lambda i: (0, 0)),
            pl.BlockSpec(Wls_h.shape, lambda i: (0, 0)),
            pl.BlockSpec(Wls_x.shape, lambda i: (0, 0)),
            pl.BlockSpec((1, Z), lambda i: (0, 0)),
        ],
        out_specs=[
            pl.BlockSpec((BN, Z), lambda i: (i, 0)),
            pl.BlockSpec((BN, Z), lambda i: (i, 0)),
        ],
        out_shape=[
            jax.ShapeDtypeStruct((N_NODES, Z), jnp.float32),
            jax.ShapeDtypeStruct((N_NODES, Z), jnp.float32),
        ],
    )(y2, deg, x, Wmu_h, Wmu_x, bmu[None], Wls_h, Wls_x, bls[None])


def kernel(x, senders, receivers, W1, b1, W2, b2, Wmu, bmu, Wls, bls):
    E = senders.shape[0]
    n = x.shape[0]
    EP = ((E + NSUB * EB - 1) // (NSUB * EB)) * (NSUB * EB)
    pad = EP - E
    pad_n = jnp.full((pad,), n, jnp.int32)
    s_deg = jnp.concatenate([senders, pad_n])
    r_pad = jnp.concatenate([receivers, pad_n])
    s_gat = jnp.concatenate([senders, jnp.zeros((pad,), jnp.int32)])

    ones_rows = jnp.ones((EB, HC), jnp.float32)
    zeros_rows = jnp.zeros((RPT, HC), jnp.float32)

    deg = _sc_degree(EP)(s_deg, r_pad, ones_rows, zeros_rows)
    z0, z1 = _tc_layer1(x, W1, b1, deg)
    y1 = _sc_message(EP)(z0, z1, s_gat, r_pad, zeros_rows)
    z20, z21 = _tc_layer2(y1, W2, b2, deg)
    y2 = _sc_message(EP)(z20, z21, s_gat, r_pad, zeros_rows)
    mu, logsig2 = _tc_heads(y2, deg, x, Wmu[:HID], Wmu[HID:], bmu,
                            Wls[:HID], Wls[HID:], bls)
    return (mu, logsig2)


# speedup vs baseline: 8.6669x; 1.8617x over previous
import functools

import jax
import jax.numpy as jnp
from jax import lax
from jax.experimental import pallas as pl
from jax.experimental.pallas import tpu as pltpu
from jax.experimental.pallas import tpu_sc as plsc

N_NODES = 100000
HID = 20
NSUB = 16
EB = 128
HC = 16
NP = N_NODES + HC
RPT = NP // NSUB
BN = 2000

_mesh = plsc.VectorSubcoreMesh(core_axis_name="c", subcore_axis_name="s")
_CP = pltpu.CompilerParams(use_tc_tiling_on_sc=False)


KCH = 17


def _sc_degree(EP):
    cpt = EP // EB // NSUB
    assert cpt % KCH == 0
    groups = cpt // KCH

    @functools.partial(
        pl.kernel,
        out_type=jax.ShapeDtypeStruct((2, NP, HC), jnp.float32),
        mesh=_mesh,
        compiler_params=_CP,
        scratch_types=[
            pltpu.VMEM((KCH, EB), jnp.int32),
            pltpu.VMEM((EB, HC), jnp.float32),
            pltpu.VMEM_SHARED((NP, HC), jnp.float32),
            pltpu.SemaphoreType.DMA,
        ],
    )
    def deg_kernel(s_hbm, r_hbm, ones_hbm, zeros_hbm, deg_out,
                   idx, ones_v, acc, sem):
        c = lax.axis_index("c")
        s = lax.axis_index("s")
        r0 = s * RPT
        pltpu.sync_copy(zeros_hbm, acc.at[pl.ds(r0, RPT)])
        pltpu.sync_copy(ones_hbm, ones_v)
        plsc.subcore_barrier()

        def scan_edges(e_hbm):
            def body(g, carry):
                pltpu.sync_copy(e_hbm.at[pl.ds(s * cpt + g * KCH, KCH)], idx)
                ds = [pltpu.async_copy(ones_v, acc.at[idx.at[j]], sem,
                                       add=True) for j in range(KCH)]
                for d in ds:
                    d.wait()
                return carry
            lax.fori_loop(0, groups, body, 0)

        @pl.when(c == 0)
        def _():
            scan_edges(s_hbm)

        @pl.when(c == 1)
        def _():
            scan_edges(r_hbm)

        plsc.subcore_barrier()
        pltpu.sync_copy(acc.at[pl.ds(r0, RPT)], deg_out.at[c, pl.ds(r0, RPT)])

    return deg_kernel


def _sc_message(EP):
    cpt = EP // EB // NSUB
    assert cpt % KCH == 0
    groups = cpt // KCH

    @functools.partial(
        pl.kernel,
        out_type=jax.ShapeDtypeStruct((2, NP, HC), jnp.float32),
        mesh=_mesh,
        compiler_params=_CP,
        scratch_types=[
            pltpu.VMEM((KCH, EB), jnp.int32),
            pltpu.VMEM((KCH, EB), jnp.int32),
            pltpu.VMEM((EB, HC), jnp.float32),
            pltpu.VMEM((EB, HC), jnp.float32),
            pltpu.VMEM_SHARED((NP, HC), jnp.float32),
            pltpu.SemaphoreType.DMA,
            pltpu.SemaphoreType.DMA,
            pltpu.SemaphoreType.DMA,
        ],
    )
    def msg_kernel(z0_hbm, z1_hbm, s_hbm, r_hbm, zeros_hbm, y_out,
                   sidx, ridx, rows0, rows1, acc, gsem, ssem0, ssem1):
        c = lax.axis_index("c")
        s = lax.axis_index("s")
        r0 = s * RPT
        rows = (rows0, rows1)
        ssem = (ssem0, ssem1)
        pltpu.sync_copy(zeros_hbm, acc.at[pl.ds(r0, RPT)])
        plsc.subcore_barrier()

        def scan_edges(z_hbm):
            def body(g, carry):
                base = s * cpt + g * KCH
                pltpu.sync_copy(s_hbm.at[pl.ds(base, KCH)], sidx)
                pltpu.sync_copy(r_hbm.at[pl.ds(base, KCH)], ridx)
                dg = pltpu.async_copy(z_hbm.at[sidx.at[0]], rows[0], gsem)
                dsc = [None, None]
                for j in range(KCH):
                    dg.wait()
                    dsc[j % 2] = pltpu.async_copy(
                        rows[j % 2], acc.at[ridx.at[j]], ssem[j % 2], add=True)
                    if j + 1 < KCH:
                        if j >= 1:
                            dsc[(j + 1) % 2].wait()
                        dg = pltpu.async_copy(z_hbm.at[sidx.at[j + 1]],
                                              rows[(j + 1) % 2], gsem)
                dsc[(KCH - 2) % 2].wait()
                dsc[(KCH - 1) % 2].wait()
                return carry
            lax.fori_loop(0, groups, body, 0)

        @pl.when(c == 0)
        def _():
            scan_edges(z0_hbm)

        @pl.when(c == 1)
        def _():
            scan_edges(z1_hbm)

        plsc.subcore_barrier()
        pltpu.sync_copy(acc.at[pl.ds(r0, RPT)], y_out.at[c, pl.ds(r0, RPT)])

    return msg_kernel


def _norm_softmax(h, deg_col):
    h = jnp.maximum(h, 0.0)
    e = jnp.exp(h - jnp.max(h, axis=-1, keepdims=True))
    sm = e / jnp.sum(e, axis=-1, keepdims=True)
    return sm * lax.rsqrt(jnp.maximum(deg_col, 1.0))[:, None]


def _split_cols(z):
    pad = jnp.zeros((z.shape[0], 2 * HC - HID), z.dtype)
    return z[:, :HC], jnp.concatenate([z[:, HC:], pad], axis=-1)


def _tc_layer1(x, W1, b1, deg):
    grid = (N_NODES // BN,)

    def body(x_ref, w_ref, b_ref, d_ref, o0_ref, o1_ref):
        h = jnp.dot(x_ref[...], w_ref[...], preferred_element_type=jnp.float32)
        z = _norm_softmax(h + b_ref[...], d_ref[0][:, 0])
        z0, z1 = _split_cols(z)
        o0_ref[...] = z0
        o1_ref[...] = z1

    return pl.pallas_call(
        body,
        grid=grid,
        in_specs=[
            pl.BlockSpec((BN, x.shape[1]), lambda i: (i, 0)),
            pl.BlockSpec(W1.shape, lambda i: (0, 0)),
            pl.BlockSpec((1, HID), lambda i: (0, 0)),
            pl.BlockSpec((2, BN, HC), lambda i: (0, i, 0)),
        ],
        out_specs=[
            pl.BlockSpec((BN, HC), lambda i: (i, 0)),
            pl.BlockSpec((BN, HC), lambda i: (i, 0)),
        ],
        out_shape=[
            jax.ShapeDtypeStruct((N_NODES, HC), jnp.float32),
            jax.ShapeDtypeStruct((N_NODES, HC), jnp.float32),
        ],
    )(x, W1, b1[None], deg)


def _recombine(y_ref, d_ref):
    rd = d_ref[1][:, 0]
    h = jnp.concatenate([y_ref[0], y_ref[1][:, : HID - HC]], axis=-1)
    return h * lax.rsqrt(jnp.maximum(rd, 1.0))[:, None]


def _tc_layer2(y1, W2, b2, deg):
    grid = (N_NODES // BN,)

    def body(y_ref, w_ref, b_ref, d_ref, o0_ref, o1_ref):
        h1 = _recombine(y_ref, d_ref)
        h = jnp.dot(h1, w_ref[...], preferred_element_type=jnp.float32)
        z = _norm_softmax(h + b_ref[...], d_ref[0][:, 0])
        z0, z1 = _split_cols(z)
        o0_ref[...] = z0
        o1_ref[...] = z1

    return pl.pallas_call(
        body,
        grid=grid,
        in_specs=[
            pl.BlockSpec((2, BN, HC), lambda i: (0, i, 0)),
            pl.BlockSpec(W2.shape, lambda i: (0, 0)),
            pl.BlockSpec((1, HID), lambda i: (0, 0)),
            pl.BlockSpec((2, BN, HC), lambda i: (0, i, 0)),
        ],
        out_specs=[
            pl.BlockSpec((BN, HC), lambda i: (i, 0)),
            pl.BlockSpec((BN, HC), lambda i: (i, 0)),
        ],
        out_shape=[
            jax.ShapeDtypeStruct((N_NODES, HC), jnp.float32),
            jax.ShapeDtypeStruct((N_NODES, HC), jnp.float32),
        ],
    )(y1, W2, b2[None], deg)


def _tc_heads(y2, deg, x, Wmu_h, Wmu_x, bmu, Wls_h, Wls_x, bls):
    grid = (N_NODES // BN,)
    Z = Wmu_h.shape[1]

    def body(y_ref, d_ref, x_ref, wmh_ref, wmx_ref, bm_ref,
             wlh_ref, wlx_ref, bl_ref, mu_ref, ls_ref):
        h2 = _recombine(y_ref, d_ref)
        xb = x_ref[...]
        mu_ref[...] = (jnp.dot(h2, wmh_ref[...], preferred_element_type=jnp.float32)
                       + jnp.dot(xb, wmx_ref[...], preferred_element_type=jnp.float32)
                       + bm_ref[...])
        ls_ref[...] = (jnp.dot(h2, wlh_ref[...], preferred_element_type=jnp.float32)
                       + jnp.dot(xb, wlx_ref[...], preferred_element_type=jnp.float32)
                       + bl_ref[...])

    return pl.pallas_call(
        body,
        grid=grid,
        in_specs=[
            pl.BlockSpec((2, BN, HC), lambda i: (0, i, 0)),
            pl.BlockSpec((2, BN, HC), lambda i: (0, i, 0)),
            pl.BlockSpec((BN, x.shape[1]), lambda i: (i, 0)),
            pl.BlockSpec(Wmu_h.shape, lambda i: (0, 0)),
            pl.BlockSpec(Wmu_x.shape, lambda i: (0, 0)),
            pl.BlockSpec((1, Z), lambda i: (0, 0)),
            pl.BlockSpec(Wls_h.shape, lambda i: (0, 0)),
            pl.BlockSpec(Wls_x.shape, lambda i: (0, 0)),
            pl.BlockSpec((1, Z), lambda i: (0, 0)),
        ],
        out_specs=[
            pl.BlockSpec((BN, Z), lambda i: (i, 0)),
            pl.BlockSpec((BN, Z), lambda i: (i, 0)),
        ],
        out_shape=[
            jax.ShapeDtypeStruct((N_NODES, Z), jnp.float32),
            jax.ShapeDtypeStruct((N_NODES, Z), jnp.float32),
        ],
    )(y2, deg, x, Wmu_h, Wmu_x, bmu[None], Wls_h, Wls_x, bls[None])


def kernel(x, senders, receivers, W1, b1, W2, b2, Wmu, bmu, Wls, bls):
    E = senders.shape[0]
    n = x.shape[0]
    quantum = NSUB * EB * KCH
    EP = ((E + quantum - 1) // quantum) * quantum
    pad = EP - E
    pad_n = jnp.full((pad,), n, jnp.int32)
    s_deg = jnp.concatenate([senders, pad_n]).reshape(-1, EB)
    r_pad = jnp.concatenate([receivers, pad_n]).reshape(-1, EB)
    s_gat = jnp.concatenate([senders, jnp.zeros((pad,), jnp.int32)]).reshape(-1, EB)

    ones_rows = jnp.ones((EB, HC), jnp.float32)
    zeros_rows = jnp.zeros((RPT, HC), jnp.float32)

    deg = _sc_degree(EP)(s_deg, r_pad, ones_rows, zeros_rows)
    z0, z1 = _tc_layer1(x, W1, b1, deg)
    y1 = _sc_message(EP)(z0, z1, s_gat, r_pad, zeros_rows)
    z20, z21 = _tc_layer2(y1, W2, b2, deg)
    y2 = _sc_message(EP)(z20, z21, s_gat, r_pad, zeros_rows)
    mu, logsig2 = _tc_heads(y2, deg, x, Wmu[:HID], Wmu[HID:], bmu,
                            Wls[:HID], Wls[HID:], bls)
    return (mu, logsig2)


# speedup vs baseline: 13.6312x; 1.5728x over previous
import functools

import jax
import jax.numpy as jnp
from jax import lax
from jax.experimental import pallas as pl
from jax.experimental.pallas import tpu as pltpu
from jax.experimental.pallas import tpu_sc as plsc

N_NODES = 100000
HID = 20
NSUB = 16
EB = 128
HC = 16
NP = N_NODES + HC
RPT = NP // NSUB
BN = 2000

_mesh = plsc.VectorSubcoreMesh(core_axis_name="c", subcore_axis_name="s")
_CP = pltpu.CompilerParams(use_tc_tiling_on_sc=False)


KCH = 34
NBUF = 4


def _sc_degree(EP):
    cpt = EP // EB // NSUB
    assert cpt % KCH == 0
    groups = cpt // KCH

    @functools.partial(
        pl.kernel,
        out_type=jax.ShapeDtypeStruct((2, NP, HC), jnp.float32),
        mesh=_mesh,
        compiler_params=_CP,
        scratch_types=[
            pltpu.VMEM((KCH, EB), jnp.int32),
            pltpu.VMEM((EB, HC), jnp.float32),
            pltpu.VMEM_SHARED((NP, HC), jnp.float32),
            pltpu.SemaphoreType.DMA,
        ],
    )
    def deg_kernel(s_hbm, r_hbm, ones_hbm, zeros_hbm, deg_out,
                   idx, ones_v, acc, sem):
        c = lax.axis_index("c")
        s = lax.axis_index("s")
        r0 = s * RPT
        pltpu.sync_copy(zeros_hbm, acc.at[pl.ds(r0, RPT)])
        pltpu.sync_copy(ones_hbm, ones_v)
        plsc.subcore_barrier()

        def scan_edges(e_hbm):
            def body(g, carry):
                pltpu.sync_copy(e_hbm.at[pl.ds(s * cpt + g * KCH, KCH)], idx)
                ds = [pltpu.async_copy(ones_v, acc.at[idx.at[j]], sem,
                                       add=True) for j in range(KCH)]
                for d in ds:
                    d.wait()
                return carry
            lax.fori_loop(0, groups, body, 0)

        @pl.when(c == 0)
        def _():
            scan_edges(s_hbm)

        @pl.when(c == 1)
        def _():
            scan_edges(r_hbm)

        plsc.subcore_barrier()
        pltpu.sync_copy(acc.at[pl.ds(r0, RPT)], deg_out.at[c, pl.ds(r0, RPT)])

    return deg_kernel


def _sc_message(EP):
    cpt = EP // EB // NSUB
    assert cpt % KCH == 0
    groups = cpt // KCH

    @functools.partial(
        pl.kernel,
        out_type=jax.ShapeDtypeStruct((2, NP, HC), jnp.float32),
        mesh=_mesh,
        compiler_params=_CP,
        scratch_types=(
            [pltpu.VMEM((KCH, EB), jnp.int32),
             pltpu.VMEM((KCH, EB), jnp.int32)]
            + [pltpu.VMEM((EB, HC), jnp.float32) for _ in range(NBUF)]
            + [pltpu.VMEM_SHARED((NP, HC), jnp.float32)]
            + [pltpu.SemaphoreType.DMA for _ in range(2 * NBUF)]
        ),
    )
    def msg_kernel(z0_hbm, z1_hbm, s_hbm, r_hbm, zeros_hbm, y_out,
                   sidx, ridx, *refs):
        rows = refs[:NBUF]
        acc = refs[NBUF]
        gsem = refs[NBUF + 1: NBUF + 1 + NBUF]
        ssem = refs[NBUF + 1 + NBUF:]
        c = lax.axis_index("c")
        s = lax.axis_index("s")
        r0 = s * RPT
        pltpu.sync_copy(zeros_hbm, acc.at[pl.ds(r0, RPT)])
        plsc.subcore_barrier()

        def scan_edges(z_hbm):
            def body(g, carry):
                base = s * cpt + g * KCH
                pltpu.sync_copy(s_hbm.at[pl.ds(base, KCH)], sidx)
                pltpu.sync_copy(r_hbm.at[pl.ds(base, KCH)], ridx)
                dgat = [None] * NBUF
                dsc = [None] * NBUF
                for b in range(NBUF - 1):
                    dgat[b] = pltpu.async_copy(z_hbm.at[sidx.at[b]],
                                               rows[b], gsem[b])
                for j in range(KCH):
                    b = j % NBUF
                    dgat[b].wait()
                    dsc[b] = pltpu.async_copy(rows[b], acc.at[ridx.at[j]],
                                              ssem[b], add=True)
                    nxt = j + NBUF - 1
                    if nxt < KCH:
                        nb = nxt % NBUF
                        if dsc[nb] is not None:
                            dsc[nb].wait()
                        dgat[nb] = pltpu.async_copy(z_hbm.at[sidx.at[nxt]],
                                                    rows[nb], gsem[nb])
                for j in range(max(0, KCH - NBUF), KCH):
                    dsc[j % NBUF].wait()
                return carry
            lax.fori_loop(0, groups, body, 0)

        @pl.when(c == 0)
        def _():
            scan_edges(z0_hbm)

        @pl.when(c == 1)
        def _():
            scan_edges(z1_hbm)

        plsc.subcore_barrier()
        pltpu.sync_copy(acc.at[pl.ds(r0, RPT)], y_out.at[c, pl.ds(r0, RPT)])

    return msg_kernel


def _norm_softmax(h, deg_col):
    h = jnp.maximum(h, 0.0)
    e = jnp.exp(h - jnp.max(h, axis=-1, keepdims=True))
    sm = e / jnp.sum(e, axis=-1, keepdims=True)
    return sm * lax.rsqrt(jnp.maximum(deg_col, 1.0))[:, None]


def _split_cols(z):
    pad = jnp.zeros((z.shape[0], 2 * HC - HID), z.dtype)
    return z[:, :HC], jnp.concatenate([z[:, HC:], pad], axis=-1)


def _tc_layer1(x, W1, b1, deg):
    grid = (N_NODES // BN,)

    def body(x_ref, w_ref, b_ref, d_ref, o0_ref, o1_ref):
        h = jnp.dot(x_ref[...], w_ref[...], preferred_element_type=jnp.float32)
        z = _norm_softmax(h + b_ref[...], d_ref[0][:, 0])
        z0, z1 = _split_cols(z)
        o0_ref[...] = z0
        o1_ref[...] = z1

    return pl.pallas_call(
        body,
        grid=grid,
        in_specs=[
            pl.BlockSpec((BN, x.shape[1]), lambda i: (i, 0)),
            pl.BlockSpec(W1.shape, lambda i: (0, 0)),
            pl.BlockSpec((1, HID), lambda i: (0, 0)),
            pl.BlockSpec((2, BN, HC), lambda i: (0, i, 0)),
        ],
        out_specs=[
            pl.BlockSpec((BN, HC), lambda i: (i, 0)),
            pl.BlockSpec((BN, HC), lambda i: (i, 0)),
        ],
        out_shape=[
            jax.ShapeDtypeStruct((N_NODES, HC), jnp.float32),
            jax.ShapeDtypeStruct((N_NODES, HC), jnp.float32),
        ],
    )(x, W1, b1[None], deg)


def _recombine(y_ref, d_ref):
    rd = d_ref[1][:, 0]
    h = jnp.concatenate([y_ref[0], y_ref[1][:, : HID - HC]], axis=-1)
    return h * lax.rsqrt(jnp.maximum(rd, 1.0))[:, None]


def _tc_layer2(y1, W2, b2, deg):
    grid = (N_NODES // BN,)

    def body(y_ref, w_ref, b_ref, d_ref, o0_ref, o1_ref):
        h1 = _recombine(y_ref, d_ref)
        h = jnp.dot(h1, w_ref[...], preferred_element_type=jnp.float32)
        z = _norm_softmax(h + b_ref[...], d_ref[0][:, 0])
        z0, z1 = _split_cols(z)
        o0_ref[...] = z0
        o1_ref[...] = z1

    return pl.pallas_call(
        body,
        grid=grid,
        in_specs=[
            pl.BlockSpec((2, BN, HC), lambda i: (0, i, 0)),
            pl.BlockSpec(W2.shape, lambda i: (0, 0)),
            pl.BlockSpec((1, HID), lambda i: (0, 0)),
            pl.BlockSpec((2, BN, HC), lambda i: (0, i, 0)),
        ],
        out_specs=[
            pl.BlockSpec((BN, HC), lambda i: (i, 0)),
            pl.BlockSpec((BN, HC), lambda i: (i, 0)),
        ],
        out_shape=[
            jax.ShapeDtypeStruct((N_NODES, HC), jnp.float32),
            jax.ShapeDtypeStruct((N_NODES, HC), jnp.float32),
        ],
    )(y1, W2, b2[None], deg)


def _tc_heads(y2, deg, x, Wmu_h, Wmu_x, bmu, Wls_h, Wls_x, bls):
    grid = (N_NODES // BN,)
    Z = Wmu_h.shape[1]

    def body(y_ref, d_ref, x_ref, wmh_ref, wmx_ref, bm_ref,
             wlh_ref, wlx_ref, bl_ref, mu_ref, ls_ref):
        h2 = _recombine(y_ref, d_ref)
        xb = x_ref[...]
        mu_ref[...] = (jnp.dot(h2, wmh_ref[...], preferred_element_type=jnp.float32)
                       + jnp.dot(xb, wmx_ref[...], preferred_element_type=jnp.float32)
                       + bm_ref[...])
        ls_ref[...] = (jnp.dot(h2, wlh_ref[...], preferred_element_type=jnp.float32)
                       + jnp.dot(xb, wlx_ref[...], preferred_element_type=jnp.float32)
                       + bl_ref[...])

    return pl.pallas_call(
        body,
        grid=grid,
        in_specs=[
            pl.BlockSpec((2, BN, HC), lambda i: (0, i, 0)),
            pl.BlockSpec((2, BN, HC), lambda i: (0, i, 0)),
            pl.BlockSpec((BN, x.shape[1]), lambda i: (i, 0)),
            pl.BlockSpec(Wmu_h.shape, lambda i: (0, 0)),
            pl.BlockSpec(Wmu_x.shape, lambda i: (0, 0)),
            pl.BlockSpec((1, Z), lambda i: (0, 0)),
            pl.BlockSpec(Wls_h.shape, lambda i: (0, 0)),
            pl.BlockSpec(Wls_x.shape, lambda i: (0, 0)),
            pl.BlockSpec((1, Z), lambda i: (0, 0)),
        ],
        out_specs=[
            pl.BlockSpec((BN, Z), lambda i: (i, 0)),
            pl.BlockSpec((BN, Z), lambda i: (i, 0)),
        ],
        out_shape=[
            jax.ShapeDtypeStruct((N_NODES, Z), jnp.float32),
            jax.ShapeDtypeStruct((N_NODES, Z), jnp.float32),
        ],
    )(y2, deg, x, Wmu_h, Wmu_x, bmu[None], Wls_h, Wls_x, bls[None])


def kernel(x, senders, receivers, W1, b1, W2, b2, Wmu, bmu, Wls, bls):
    E = senders.shape[0]
    n = x.shape[0]
    quantum = NSUB * EB * KCH
    EP = ((E + quantum - 1) // quantum) * quantum
    pad = EP - E
    pad_n = jnp.full((pad,), n, jnp.int32)
    s_deg = jnp.concatenate([senders, pad_n]).reshape(-1, EB)
    r_pad = jnp.concatenate([receivers, pad_n]).reshape(-1, EB)
    s_gat = jnp.concatenate([senders, jnp.zeros((pad,), jnp.int32)]).reshape(-1, EB)

    ones_rows = jnp.ones((EB, HC), jnp.float32)
    zeros_rows = jnp.zeros((RPT, HC), jnp.float32)

    deg = _sc_degree(EP)(s_deg, r_pad, ones_rows, zeros_rows)
    z0, z1 = _tc_layer1(x, W1, b1, deg)
    y1 = _sc_message(EP)(z0, z1, s_gat, r_pad, zeros_rows)
    z20, z21 = _tc_layer2(y1, W2, b2, deg)
    y2 = _sc_message(EP)(z20, z21, s_gat, r_pad, zeros_rows)
    mu, logsig2 = _tc_heads(y2, deg, x, Wmu[:HID], Wmu[HID:], bmu,
                            Wls[:HID], Wls[HID:], bls)
    return (mu, logsig2)


# speedup vs baseline: 15.5096x; 1.1378x over previous
import functools

import jax
import jax.numpy as jnp
from jax import lax
from jax.experimental import pallas as pl
from jax.experimental.pallas import tpu as pltpu
from jax.experimental.pallas import tpu_sc as plsc

N_NODES = 100000
HID = 20
NSUB = 16
EB = 128
HC = 16
NP = N_NODES + HC
RPT = NP // NSUB
BN = 2000

_mesh = plsc.VectorSubcoreMesh(core_axis_name="c", subcore_axis_name="s")
_CP = pltpu.CompilerParams(use_tc_tiling_on_sc=False)


KCH = 23
NBUF = 6


def _sc_degree(EP):
    cpt = EP // EB // NSUB
    assert cpt % KCH == 0
    groups = cpt // KCH

    @functools.partial(
        pl.kernel,
        out_type=jax.ShapeDtypeStruct((2, NP, HC), jnp.float32),
        mesh=_mesh,
        compiler_params=_CP,
        scratch_types=[
            pltpu.VMEM((KCH, EB), jnp.int32),
            pltpu.VMEM((EB, HC), jnp.float32),
            pltpu.VMEM_SHARED((NP, HC), jnp.float32),
            pltpu.SemaphoreType.DMA,
        ],
    )
    def deg_kernel(s_hbm, r_hbm, ones_hbm, zeros_hbm, deg_out,
                   idx, ones_v, acc, sem):
        c = lax.axis_index("c")
        s = lax.axis_index("s")
        r0 = s * RPT
        pltpu.sync_copy(zeros_hbm, acc.at[pl.ds(r0, RPT)])
        pltpu.sync_copy(ones_hbm, ones_v)
        plsc.subcore_barrier()

        def scan_edges(e_hbm):
            def body(g, carry):
                pltpu.sync_copy(e_hbm.at[pl.ds(s * cpt + g * KCH, KCH)], idx)
                ds = [pltpu.async_copy(ones_v, acc.at[idx.at[j]], sem,
                                       add=True) for j in range(KCH)]
                for d in ds:
                    d.wait()
                return carry
            lax.fori_loop(0, groups, body, 0)

        @pl.when(c == 0)
        def _():
            scan_edges(s_hbm)

        @pl.when(c == 1)
        def _():
            scan_edges(r_hbm)

        plsc.subcore_barrier()
        pltpu.sync_copy(acc.at[pl.ds(r0, RPT)], deg_out.at[c, pl.ds(r0, RPT)])

    return deg_kernel


def _sc_message(EP):
    cpt = EP // EB // NSUB
    assert cpt % (2 * KCH) == 0
    half_groups = cpt // KCH // 2

    @functools.partial(
        pl.kernel,
        out_type=jax.ShapeDtypeStruct((2, NP, HC), jnp.float32),
        mesh=_mesh,
        compiler_params=_CP,
        scratch_types=(
            [pltpu.VMEM((KCH, EB), jnp.int32) for _ in range(4)]
            + [pltpu.VMEM((EB, HC), jnp.float32) for _ in range(NBUF)]
            + [pltpu.VMEM_SHARED((NP, HC), jnp.float32)]
            + [pltpu.SemaphoreType.DMA for _ in range(2 * NBUF + 4)]
        ),
    )
    def msg_kernel(z0_hbm, z1_hbm, s_hbm, r_hbm, zeros_hbm, y_out, *refs):
        sidx = refs[0:2]
        ridx = refs[2:4]
        rows = refs[4:4 + NBUF]
        acc = refs[4 + NBUF]
        gsem = refs[5 + NBUF: 5 + 2 * NBUF]
        ssem = refs[5 + 2 * NBUF: 5 + 3 * NBUF]
        isem = refs[5 + 3 * NBUF:]
        c = lax.axis_index("c")
        s = lax.axis_index("s")
        r0 = s * RPT
        pltpu.sync_copy(zeros_hbm, acc.at[pl.ds(r0, RPT)])
        plsc.subcore_barrier()

        def fire_idx(base, p):
            pltpu.async_copy(s_hbm.at[pl.ds(base, KCH)], sidx[p], isem[p])
            pltpu.async_copy(r_hbm.at[pl.ds(base, KCH)], ridx[p], isem[2 + p])

        def wait_idx(base, p):
            pltpu.make_async_copy(s_hbm.at[pl.ds(base, KCH)], sidx[p],
                                  isem[p]).wait()
            pltpu.make_async_copy(r_hbm.at[pl.ds(base, KCH)], ridx[p],
                                  isem[2 + p]).wait()

        def process(z_hbm, sb, rb):
            dgat = [None] * NBUF
            dsc = [None] * NBUF
            for b in range(NBUF - 1):
                dgat[b] = pltpu.async_copy(z_hbm.at[sb.at[b]],
                                           rows[b], gsem[b])
            for j in range(KCH):
                b = j % NBUF
                dgat[b].wait()
                dsc[b] = pltpu.async_copy(rows[b], acc.at[rb.at[j]],
                                          ssem[b], add=True)
                nxt = j + NBUF - 1
                if nxt < KCH:
                    nb = nxt % NBUF
                    if dsc[nb] is not None:
                        dsc[nb].wait()
                    dgat[nb] = pltpu.async_copy(z_hbm.at[sb.at[nxt]],
                                                rows[nb], gsem[nb])
            for j in range(max(0, KCH - NBUF), KCH):
                dsc[j % NBUF].wait()

        def scan_edges(z_hbm):
            fire_idx(s * cpt, 0)

            def body(t, carry):
                base0 = s * cpt + 2 * t * KCH
                wait_idx(base0, 0)
                fire_idx(base0 + KCH, 1)
                process(z_hbm, sidx[0], ridx[0])
                wait_idx(base0 + KCH, 1)

                @pl.when(t + 1 < half_groups)
                def _():
                    fire_idx(base0 + 2 * KCH, 0)

                process(z_hbm, sidx[1], ridx[1])
                return carry
            lax.fori_loop(0, half_groups, body, 0)

        @pl.when(c == 0)
        def _():
            scan_edges(z0_hbm)

        @pl.when(c == 1)
        def _():
            scan_edges(z1_hbm)

        plsc.subcore_barrier()
        pltpu.sync_copy(acc.at[pl.ds(r0, RPT)], y_out.at[c, pl.ds(r0, RPT)])

    return msg_kernel


def _norm_softmax(h, deg_col):
    h = jnp.maximum(h, 0.0)
    e = jnp.exp(h - jnp.max(h, axis=-1, keepdims=True))
    sm = e / jnp.sum(e, axis=-1, keepdims=True)
    return sm * lax.rsqrt(jnp.maximum(deg_col, 1.0))[:, None]


def _split_cols(z):
    pad = jnp.zeros((z.shape[0], 2 * HC - HID), z.dtype)
    return z[:, :HC], jnp.concatenate([z[:, HC:], pad], axis=-1)


def _tc_layer1(x, W1, b1, deg):
    grid = (N_NODES // BN,)

    def body(x_ref, w_ref, b_ref, d_ref, o0_ref, o1_ref):
        h = jnp.dot(x_ref[...], w_ref[...], preferred_element_type=jnp.float32)
        z = _norm_softmax(h + b_ref[...], d_ref[0][:, 0])
        z0, z1 = _split_cols(z)
        o0_ref[...] = z0
        o1_ref[...] = z1

    return pl.pallas_call(
        body,
        grid=grid,
        in_specs=[
            pl.BlockSpec((BN, x.shape[1]), lambda i: (i, 0)),
            pl.BlockSpec(W1.shape, lambda i: (0, 0)),
            pl.BlockSpec((1, HID), lambda i: (0, 0)),
            pl.BlockSpec((2, BN, HC), lambda i: (0, i, 0)),
        ],
        out_specs=[
            pl.BlockSpec((BN, HC), lambda i: (i, 0)),
            pl.BlockSpec((BN, HC), lambda i: (i, 0)),
        ],
        out_shape=[
            jax.ShapeDtypeStruct((N_NODES, HC), jnp.float32),
            jax.ShapeDtypeStruct((N_NODES, HC), jnp.float32),
        ],
    )(x, W1, b1[None], deg)


def _recombine(y_ref, d_ref):
    rd = d_ref[1][:, 0]
    h = jnp.concatenate([y_ref[0], y_ref[1][:, : HID - HC]], axis=-1)
    return h * lax.rsqrt(jnp.maximum(rd, 1.0))[:, None]


def _tc_layer2(y1, W2, b2, deg):
    grid = (N_NODES // BN,)

    def body(y_ref, w_ref, b_ref, d_ref, o0_ref, o1_ref):
        h1 = _recombine(y_ref, d_ref)
        h = jnp.dot(h1, w_ref[...], preferred_element_type=jnp.float32)
        z = _norm_softmax(h + b_ref[...], d_ref[0][:, 0])
        z0, z1 = _split_cols(z)
        o0_ref[...] = z0
        o1_ref[...] = z1

    return pl.pallas_call(
        body,
        grid=grid,
        in_specs=[
            pl.BlockSpec((2, BN, HC), lambda i: (0, i, 0)),
            pl.BlockSpec(W2.shape, lambda i: (0, 0)),
            pl.BlockSpec((1, HID), lambda i: (0, 0)),
            pl.BlockSpec((2, BN, HC), lambda i: (0, i, 0)),
        ],
        out_specs=[
            pl.BlockSpec((BN, HC), lambda i: (i, 0)),
            pl.BlockSpec((BN, HC), lambda i: (i, 0)),
        ],
        out_shape=[
            jax.ShapeDtypeStruct((N_NODES, HC), jnp.float32),
            jax.ShapeDtypeStruct((N_NODES, HC), jnp.float32),
        ],
    )(y1, W2, b2[None], deg)


def _tc_heads(y2, deg, x, Wmu_h, Wmu_x, bmu, Wls_h, Wls_x, bls):
    grid = (N_NODES // BN,)
    Z = Wmu_h.shape[1]

    def body(y_ref, d_ref, x_ref, wmh_ref, wmx_ref, bm_ref,
             wlh_ref, wlx_ref, bl_ref, mu_ref, ls_ref):
        h2 = _recombine(y_ref, d_ref)
        xb = x_ref[...]
        mu_ref[...] = (jnp.dot(h2, wmh_ref[...], preferred_element_type=jnp.float32)
                       + jnp.dot(xb, wmx_ref[...], preferred_element_type=jnp.float32)
                       + bm_ref[...])
        ls_ref[...] = (jnp.dot(h2, wlh_ref[...], preferred_element_type=jnp.float32)
                       + jnp.dot(xb, wlx_ref[...], preferred_element_type=jnp.float32)
                       + bl_ref[...])

    return pl.pallas_call(
        body,
        grid=grid,
        in_specs=[
            pl.BlockSpec((2, BN, HC), lambda i: (0, i, 0)),
            pl.BlockSpec((2, BN, HC), lambda i: (0, i, 0)),
            pl.BlockSpec((BN, x.shape[1]), lambda i: (i, 0)),
            pl.BlockSpec(Wmu_h.shape, lambda i: (0, 0)),
            pl.BlockSpec(Wmu_x.shape, lambda i: (0, 0)),
            pl.BlockSpec((1, Z), lambda i: (0, 0)),
            pl.BlockSpec(Wls_h.shape, lambda i: (0, 0)),
            pl.BlockSpec(Wls_x.shape, lambda i: (0, 0)),
            pl.BlockSpec((1, Z), lambda i: (0, 0)),
        ],
        out_specs=[
            pl.BlockSpec((BN, Z), lambda i: (i, 0)),
            pl.BlockSpec((BN, Z), lambda i: (i, 0)),
        ],
        out_shape=[
            jax.ShapeDtypeStruct((N_NODES, Z), jnp.float32),
            jax.ShapeDtypeStruct((N_NODES, Z), jnp.float32),
        ],
    )(y2, deg, x, Wmu_h, Wmu_x, bmu[None], Wls_h, Wls_x, bls[None])


def kernel(x, senders, receivers, W1, b1, W2, b2, Wmu, bmu, Wls, bls):
    E = senders.shape[0]
    n = x.shape[0]
    quantum = NSUB * EB * KCH * 2
    EP = ((E + quantum - 1) // quantum) * quantum
    pad = EP - E
    pad_n = jnp.full((pad,), n, jnp.int32)
    s_deg = jnp.concatenate([senders, pad_n]).reshape(-1, EB)
    r_pad = jnp.concatenate([receivers, pad_n]).reshape(-1, EB)
    s_gat = jnp.concatenate([senders, jnp.zeros((pad,), jnp.int32)]).reshape(-1, EB)

    ones_rows = jnp.ones((EB, HC), jnp.float32)
    zeros_rows = jnp.zeros((RPT, HC), jnp.float32)

    deg = _sc_degree(EP)(s_deg, r_pad, ones_rows, zeros_rows)
    z0, z1 = _tc_layer1(x, W1, b1, deg)
    y1 = _sc_message(EP)(z0, z1, s_gat, r_pad, zeros_rows)
    z20, z21 = _tc_layer2(y1, W2, b2, deg)
    y2 = _sc_message(EP)(z20, z21, s_gat, r_pad, zeros_rows)
    mu, logsig2 = _tc_heads(y2, deg, x, Wmu[:HID], Wmu[HID:], bmu,
                            Wls[:HID], Wls[HID:], bls)
    return (mu, logsig2)
